# speedup vs baseline: 2.3146x; 2.3146x over previous
import functools

import jax
import jax.numpy as jnp
from jax import lax
from jax.experimental import pallas as pl
from jax.experimental.pallas import tpu as pltpu
from jax.experimental.pallas import tpu_sc as plsc

DIM = 1024
NC = 2
NS = 16
NW = NC * NS
CHUNK = 32


def _make_gather(B: int, D: int):
  b_per_w = B // NW
  n_chunks = b_per_w // CHUNK
  mesh = plsc.VectorSubcoreMesh(core_axis_name="c", subcore_axis_name="s")

  @functools.partial(
      pl.kernel,
      mesh=mesh,
      out_type=jax.ShapeDtypeStruct((B, D), jnp.float32),
      scratch_types=[
          pltpu.VMEM((b_per_w,), jnp.int32),
          pltpu.VMEM((CHUNK, D), jnp.float32),
          pltpu.VMEM((CHUNK, D), jnp.float32),
          pltpu.SemaphoreType.DMA,
          pltpu.SemaphoreType.DMA,
      ],
  )
  def k(table_hbm, idx_hbm, out_hbm, idx_v, rows0, rows1, sem0, sem1):
    wid = lax.axis_index("s") * NC + lax.axis_index("c")
    base = wid * b_per_w
    pltpu.sync_copy(idx_hbm.at[pl.ds(base, b_per_w)], idx_v)

    bufs = (rows0, rows1)
    sems = (sem0, sem1)

    def start_gather(c, buf, sem):
      pltpu.async_copy(
          table_hbm.at[idx_v.at[pl.ds(c * CHUNK, CHUNK)]], buf, sem)

    def wait_gather(buf, sem):
      pltpu.make_async_copy(
          table_hbm.at[idx_v.at[pl.ds(0, CHUNK)]], buf, sem).wait()

    def scatter(c, buf):
      pltpu.sync_copy(buf, out_hbm.at[pl.ds(base + c * CHUNK, CHUNK)])

    start_gather(0, rows0, sem0)
    start_gather(1, rows1, sem1)

    def body(c0):
      for b in range(2):
        c = c0 + b
        wait_gather(bufs[b], sems[b])
        scatter(c, bufs[b])
        start_gather(c + 2, bufs[b], sems[b])

    pl.loop(0, n_chunks - 2, step=2, unroll=True)(body)

    for b in range(2):
      c = n_chunks - 2 + b
      wait_gather(bufs[b], sems[b])
      scatter(c, bufs[b])

  return k


def kernel(tok_idx, embeddings):
  bsz, seqlen = tok_idx.shape
  flat_idx = tok_idx.reshape(bsz * seqlen)
  out = _make_gather(bsz * seqlen, DIM)(embeddings, flat_idx)
  return out.reshape(bsz, seqlen, DIM)
